# baseline (speedup 1.0000x reference)
import functools

import jax
import jax.numpy as jnp
from jax import lax
from jax.experimental import pallas as pl
from jax.experimental.pallas import tpu as pltpu
from jax.experimental.pallas import tpu_sc as plsc

EMBED_DIM = 64
SEQ = 200
BATCH = 4096
NUM_WORKERS = 32
ROWS_PER_W = BATCH // NUM_WORKERS
LANES = 16
DCH = EMBED_DIM // LANES
WIDE = 2 * EMBED_DIM
SLOTS = 224
SLOTSB = 240
NCHUNK = SLOTS // LANES
RCHUNK = 13
CHUNK0 = 128
CHUNK1 = SLOTS - CHUNK0
NBUF = 2


def _sc_pool(text, table_wide):
    mesh = plsc.VectorSubcoreMesh(core_axis_name="c", subcore_axis_name="s")

    @functools.partial(
        pl.kernel,
        out_type=jax.ShapeDtypeStruct((BATCH, EMBED_DIM), jnp.float32),
        mesh=mesh,
        compiler_params=pltpu.CompilerParams(
            use_tc_tiling_on_sc=False, needs_layout_passes=False),
        scratch_types=[
            pltpu.VMEM((ROWS_PER_W, RCHUNK * LANES), jnp.int32),
            pltpu.VMEM((ROWS_PER_W * SLOTSB,), jnp.int32),
            pltpu.VMEM((NBUF, SLOTS, WIDE), jnp.float32),
            pltpu.VMEM((ROWS_PER_W, EMBED_DIM), jnp.float32),
            pltpu.SMEM((ROWS_PER_W,), jnp.int32),
            [pltpu.SemaphoreType.DMA] * NBUF,
        ],
    )
    def k(text_hbm, table_hbm, out_hbm, idx_v, part_v, rows_v, out_v,
          ne_smem, sems):
        wid = lax.axis_index("s") * 2 + lax.axis_index("c")
        base = wid * ROWS_PER_W
        pltpu.sync_copy(text_hbm.at[pl.ds(base, ROWS_PER_W)],
                        idx_v.at[pl.ds(0, ROWS_PER_W), pl.ds(0, SEQ)])

        lane = lax.iota(jnp.int32, 16)
        zeros_i = jnp.zeros((LANES,), jnp.int32)

        def partition_body(r, _):
            rbase = r * SLOTSB
            for c in range(NCHUNK):
                part_v[pl.ds(rbase + c * LANES, LANES)] = zeros_i

            ne = jnp.int32(0)
            for c in range(RCHUNK):
                v = idx_v[r, pl.ds(c * LANES, LANES)]
                sel = (v & 1) == 0
                if c == RCHUNK - 1:
                    sel = jnp.logical_and(sel, lane < (SEQ - c * LANES))
                ne = ne + plsc.all_reduce_population_count(sel)[0]
            ne_smem[r] = ne
            ne16 = (ne + 15) & ~15

            pe = jnp.int32(0)
            po = ne16
            for c in range(RCHUNK):
                v = idx_v[r, pl.ds(c * LANES, LANES)]
                vh = lax.shift_right_logical(v, 1)
                podd = (v & 1) == 1
                if c == RCHUNK - 1:
                    valid = lane < (SEQ - c * LANES)
                    podd_v = jnp.logical_and(podd, valid)
                    peven_v = jnp.logical_and(jnp.logical_not(podd), valid)
                else:
                    valid = None
                    podd_v = podd
                    peven_v = jnp.logical_not(podd)
                ev_i = peven_v.astype(jnp.int32)
                od_i = podd_v.astype(jnp.int32)
                excl_e = jnp.cumsum(ev_i) - ev_i
                excl_o = jnp.cumsum(od_i) - od_i
                pos = jnp.where(podd, po + excl_o, pe + excl_e)
                if valid is not None:
                    pos = jnp.where(valid, pos, SLOTS + lane)
                plsc.store_scatter(part_v, [rbase + pos], vh)
                pe = pe + plsc.all_reduce_population_count(peven_v)[0]
                po = po + plsc.all_reduce_population_count(podd_v)[0]
            return 0

        lax.fori_loop(0, ROWS_PER_W, partition_body, 0)

        scale = jnp.float32(1.0 / SEQ)

        def fire(r, s):
            rbase = r * SLOTSB
            pltpu.async_copy(
                table_hbm.at[part_v.at[pl.ds(rbase, CHUNK0)]],
                rows_v.at[s, pl.ds(0, CHUNK0)], sems[s])
            pltpu.async_copy(
                table_hbm.at[part_v.at[pl.ds(rbase + CHUNK0, CHUNK1)]],
                rows_v.at[s, pl.ds(CHUNK0, CHUNK1)], sems[s])

        def drain(s):
            pltpu.make_async_copy(
                table_hbm.at[pl.ds(0, SLOTS)], rows_v.at[s], sems[s]).wait()

        def reduce_slot(s, r):
            ne = ne_smem[r]
            ne16 = (ne + 15) & ~15
            nlo = lax.shift_right_logical(ne16, 4)

            def chunk_body(half_off):
                def body(it, acc):
                    j0 = it * LANES
                    for jj in range(LANES):
                        acc = tuple(
                            acc[k_] + rows_v[s, j0 + jj,
                                             pl.ds(half_off + k_ * LANES,
                                                   LANES)]
                            for k_ in range(DCH))
                    return acc
                return body

            zf = tuple(jnp.zeros((LANES,), jnp.float32) for _ in range(DCH))
            acc = lax.fori_loop(0, nlo, chunk_body(0), zf)
            acc = lax.fori_loop(nlo, NCHUNK, chunk_body(EMBED_DIM), acc)
            npad = (SLOTS - ne16) - (SEQ - ne)
            npf = npad.astype(jnp.float32)
            for k_ in range(DCH):
                pad_hi = rows_v[s, SLOTS - 1,
                                pl.ds(EMBED_DIM + k_ * LANES, LANES)]
                out_v[r, pl.ds(k_ * LANES, LANES)] = (
                    acc[k_] - npf * pad_hi) * scale

        for s in range(NBUF):
            fire(s, s)

        def group_body(g, _):
            rg = g * NBUF
            for s in range(NBUF):
                drain(s)
                reduce_slot(s, rg + s)
                fire(rg + s + NBUF, s)
            return 0

        lax.fori_loop(0, ROWS_PER_W // NBUF - 1, group_body, 0)
        rg = ROWS_PER_W - NBUF
        for s in range(NBUF):
            drain(s)
            reduce_slot(s, rg + s)

        pltpu.sync_copy(out_v, out_hbm.at[pl.ds(base, ROWS_PER_W)])

    return k(text, table_wide)


def _tc_head(pooled, W, b):
    def body(p_ref, w_ref, b_ref, o_ref):
        p = p_ref[...]
        w = w_ref[...]
        logits = lax.dot_general(
            p, w, dimension_numbers=(((1,), (1,)), ((), ())),
            preferred_element_type=jnp.float32)
        logits = logits + b_ref[...]
        m = jnp.max(logits, axis=1, keepdims=True)
        lse = m + jnp.log(jnp.sum(jnp.exp(logits - m), axis=1, keepdims=True))
        o_ref[...] = logits - lse

    return pl.pallas_call(
        body,
        out_shape=jax.ShapeDtypeStruct((BATCH, 2), jnp.float32),
    )(pooled, W, b.reshape(1, 2))


def kernel(text, table, W, b):
    table_wide = table.reshape(table.shape[0] // 2, WIDE)
    pooled = _sc_pool(text.astype(jnp.int32), table_wide)
    return _tc_head(pooled, W, b)


# speedup vs baseline: 5.2680x; 5.2680x over previous
import functools

import jax
import jax.numpy as jnp
from jax import lax
from jax.experimental import pallas as pl
from jax.experimental.pallas import tpu as pltpu
from jax.experimental.pallas import tpu_sc as plsc

EMBED_DIM = 64
SEQ = 200
BATCH = 4096
NUM_WORKERS = 32
ROWS_PER_W = BATCH // NUM_WORKERS
LANES = 16
DCH = EMBED_DIM // LANES
WIDE = 2 * EMBED_DIM
RCHUNK = 13
SLOTS = 216
FILLC = 14
SLOTSB = 240
TRASH = FILLC * LANES
CHUNK0 = 128
CHUNK1 = SLOTS - CHUNK0
NBUF = 2


def _sc_pool(text, table_wide):
    mesh = plsc.VectorSubcoreMesh(core_axis_name="c", subcore_axis_name="s")

    @functools.partial(
        pl.kernel,
        out_type=jax.ShapeDtypeStruct((BATCH, EMBED_DIM), jnp.float32),
        mesh=mesh,
        compiler_params=pltpu.CompilerParams(
            use_tc_tiling_on_sc=False, needs_layout_passes=False),
        scratch_types=[
            pltpu.VMEM((ROWS_PER_W, RCHUNK * LANES), jnp.int32),
            pltpu.VMEM((ROWS_PER_W, SLOTSB), jnp.int32),
            pltpu.VMEM((NBUF, FILLC * LANES, WIDE), jnp.float32),
            pltpu.VMEM((ROWS_PER_W, EMBED_DIM), jnp.float32),
            pltpu.SMEM((ROWS_PER_W,), jnp.int32),
            [pltpu.SemaphoreType.DMA] * NBUF,
        ],
    )
    def k(text_hbm, table_hbm, out_hbm, idx_v, part_v, rows_v, out_v,
          ne_smem, sems):
        wid = lax.axis_index("s") * 2 + lax.axis_index("c")
        base = wid * ROWS_PER_W
        pltpu.sync_copy(text_hbm.at[pl.ds(base, ROWS_PER_W)],
                        idx_v.at[pl.ds(0, ROWS_PER_W), pl.ds(0, SEQ)])

        lane = lax.iota(jnp.int32, 16)

        def partition_body(r, _):
            v0h = lax.shift_right_logical(idx_v[r, pl.ds(0, LANES)], 1)
            for c in range(RCHUNK):
                v = idx_v[r, pl.ds(c * LANES, LANES)]
                vh = lax.shift_right_logical(v, 1)
                if c == RCHUNK - 1:
                    vh = jnp.where(lane < (SEQ - c * LANES), vh, v0h)
                part_v[r, pl.ds(c * LANES, LANES)] = vh
            part_v[r, pl.ds(RCHUNK * LANES, LANES)] = v0h

            ne = jnp.int32(0)
            for c in range(RCHUNK):
                v = idx_v[r, pl.ds(c * LANES, LANES)]
                sel = (v & 1) == 0
                if c == RCHUNK - 1:
                    sel = jnp.logical_and(sel, lane < (SEQ - c * LANES))
                ne = ne + plsc.all_reduce_population_count(sel)[0]
            ne_smem[r] = ne
            ne16 = (ne + 15) & ~15

            pe = jnp.int32(0)
            po = ne16
            for c in range(RCHUNK):
                v = idx_v[r, pl.ds(c * LANES, LANES)]
                vh = lax.shift_right_logical(v, 1)
                podd = (v & 1) == 1
                if c == RCHUNK - 1:
                    valid = lane < (SEQ - c * LANES)
                    podd_v = jnp.logical_and(podd, valid)
                    peven_v = jnp.logical_and(jnp.logical_not(podd), valid)
                else:
                    valid = None
                    podd_v = podd
                    peven_v = jnp.logical_not(podd)
                ev_i = peven_v.astype(jnp.int32)
                od_i = podd_v.astype(jnp.int32)
                excl_e = jnp.cumsum(ev_i) - ev_i
                excl_o = jnp.cumsum(od_i) - od_i
                pos = jnp.where(podd, po + excl_o, pe + excl_e)
                if valid is not None:
                    pos = jnp.where(valid, pos, TRASH + lane)
                rvec = jnp.full((LANES,), r, jnp.int32)
                plsc.store_scatter(part_v, [rvec, pos], vh)
                pe = pe + plsc.all_reduce_population_count(peven_v)[0]
                po = po + plsc.all_reduce_population_count(podd_v)[0]
            return 0

        lax.fori_loop(0, ROWS_PER_W, partition_body, 0)

        scale = jnp.float32(1.0 / SEQ)

        def fire(r, s):
            pltpu.async_copy(
                table_hbm.at[part_v.at[r, pl.ds(0, CHUNK0)]],
                rows_v.at[s, pl.ds(0, CHUNK0)], sems[s])
            pltpu.async_copy(
                table_hbm.at[part_v.at[r, pl.ds(CHUNK0, CHUNK1)]],
                rows_v.at[s, pl.ds(CHUNK0, CHUNK1)], sems[s])

        def drain(s):
            pltpu.make_async_copy(
                table_hbm.at[pl.ds(0, SLOTS)],
                rows_v.at[s, pl.ds(0, SLOTS)], sems[s]).wait()

        def reduce_slot(s, r):
            ne = ne_smem[r]
            no = SEQ - ne
            c_lo = lax.shift_right_logical(ne, 4)
            rem_lo = ne & 15
            c_hi0 = lax.shift_right_logical(ne + 15, 4)
            n_hi = lax.shift_right_logical(no, 4)
            rem_hi = no & 15

            def chunk_body(half_off):
                def body(it, acc):
                    j0 = it * LANES
                    for jj in range(LANES):
                        acc = tuple(
                            acc[k_] + rows_v[s, j0 + jj,
                                             pl.ds(half_off + k_ * LANES,
                                                   LANES)]
                            for k_ in range(DCH))
                    return acc
                return body

            def boundary(j0, rem, half_off, acc):
                zero = jnp.zeros((LANES,), jnp.float32)
                for jj in range(LANES):
                    take = jj < rem
                    acc = tuple(
                        acc[k_] + jnp.where(
                            take,
                            rows_v[s, j0 + jj,
                                   pl.ds(half_off + k_ * LANES, LANES)],
                            zero)
                        for k_ in range(DCH))
                return acc

            zf = tuple(jnp.zeros((LANES,), jnp.float32) for _ in range(DCH))
            acc = lax.fori_loop(0, c_lo, chunk_body(0), zf)
            acc = boundary(c_lo * LANES, rem_lo, 0, acc)
            acc = lax.fori_loop(c_hi0, c_hi0 + n_hi, chunk_body(EMBED_DIM),
                                acc)
            acc = boundary((c_hi0 + n_hi) * LANES, rem_hi, EMBED_DIM, acc)
            for k_ in range(DCH):
                out_v[r, pl.ds(k_ * LANES, LANES)] = acc[k_] * scale

        for s in range(NBUF):
            fire(s, s)

        def group_body(g, _):
            rg = g * NBUF
            for s in range(NBUF):
                drain(s)
                reduce_slot(s, rg + s)
                fire(rg + s + NBUF, s)
            return 0

        lax.fori_loop(0, ROWS_PER_W // NBUF - 1, group_body, 0)
        rg = ROWS_PER_W - NBUF
        for s in range(NBUF):
            drain(s)
            reduce_slot(s, rg + s)

        pltpu.sync_copy(out_v, out_hbm.at[pl.ds(base, ROWS_PER_W)])

    return k(text, table_wide)


def _tc_head(pooled, W, b):
    def body(p_ref, w_ref, b_ref, o_ref):
        p = p_ref[...]
        w = w_ref[...]
        logits = lax.dot_general(
            p, w, dimension_numbers=(((1,), (1,)), ((), ())),
            preferred_element_type=jnp.float32)
        logits = logits + b_ref[...]
        m = jnp.max(logits, axis=1, keepdims=True)
        lse = m + jnp.log(jnp.sum(jnp.exp(logits - m), axis=1, keepdims=True))
        o_ref[...] = logits - lse

    return pl.pallas_call(
        body,
        out_shape=jax.ShapeDtypeStruct((BATCH, 2), jnp.float32),
    )(pooled, W, b.reshape(1, 2))


def kernel(text, table, W, b):
    table_wide = table.reshape(table.shape[0] // 2, WIDE)
    pooled = _sc_pool(text.astype(jnp.int32), table_wide)
    return _tc_head(pooled, W, b)


# speedup vs baseline: 6.6030x; 1.2534x over previous
import functools

import jax
import jax.numpy as jnp
from jax import lax
from jax.experimental import pallas as pl
from jax.experimental.pallas import tpu as pltpu
from jax.experimental.pallas import tpu_sc as plsc

EMBED_DIM = 64
SEQ = 200
BATCH = 4096
NUM_WORKERS = 32
ROWS_PER_W = BATCH // NUM_WORKERS
LANES = 16
DCH = EMBED_DIM // LANES
CHUNK0 = 128
CHUNK1 = SEQ - CHUNK0


NBUF = 4


def _sc_pool(text, table):
    mesh = plsc.VectorSubcoreMesh(core_axis_name="c", subcore_axis_name="s")

    @functools.partial(
        pl.kernel,
        out_type=jax.ShapeDtypeStruct((BATCH, EMBED_DIM), jnp.float32),
        mesh=mesh,
        compiler_params=pltpu.CompilerParams(use_tc_tiling_on_sc=False),
        scratch_types=[
            pltpu.VMEM((ROWS_PER_W, SEQ), jnp.int32),
            pltpu.VMEM((NBUF, SEQ, EMBED_DIM), jnp.float32),
            pltpu.VMEM((ROWS_PER_W, EMBED_DIM), jnp.float32),
            [pltpu.SemaphoreType.DMA] * NBUF,
        ],
    )
    def k(text_hbm, table_hbm, out_hbm, idx_v, rows_v, out_v, sems):
        wid = lax.axis_index("s") * 2 + lax.axis_index("c")
        base = wid * ROWS_PER_W
        pltpu.sync_copy(text_hbm.at[pl.ds(base, ROWS_PER_W)], idx_v)

        scale = jnp.float32(1.0 / SEQ)

        def fire(r, s):
            pltpu.async_copy(
                table_hbm.at[idx_v.at[r, pl.ds(0, CHUNK0)]],
                rows_v.at[s, pl.ds(0, CHUNK0)], sems[s])
            pltpu.async_copy(
                table_hbm.at[idx_v.at[r, pl.ds(CHUNK0, CHUNK1)]],
                rows_v.at[s, pl.ds(CHUNK0, CHUNK1)], sems[s])

        def drain(s):
            pltpu.make_async_copy(
                table_hbm.at[pl.ds(0, SEQ)], rows_v.at[s], sems[s]).wait()

        def reduce_slot(s, r):
            def seq_body(j, acc):
                return tuple(
                    acc[k_] + rows_v[s, j, pl.ds(k_ * LANES, LANES)]
                    for k_ in range(DCH))

            zeros = tuple(jnp.zeros((LANES,), jnp.float32) for _ in range(DCH))
            acc = lax.fori_loop(0, SEQ, seq_body, zeros, unroll=2)
            for k_ in range(DCH):
                out_v[r, pl.ds(k_ * LANES, LANES)] = acc[k_] * scale

        for s in range(NBUF):
            fire(s, s)

        def group_body(g, _):
            rg = g * NBUF
            for s in range(NBUF):
                drain(s)
                reduce_slot(s, rg + s)
                fire(rg + s + NBUF, s)
            return 0

        lax.fori_loop(0, ROWS_PER_W // NBUF - 1, group_body, 0)
        rg = ROWS_PER_W - NBUF
        for s in range(NBUF):
            drain(s)
            reduce_slot(s, rg + s)

        pltpu.sync_copy(out_v, out_hbm.at[pl.ds(base, ROWS_PER_W)])

    return k(text, table)


def _tc_head(pooled, W, b):
    def body(p_ref, w_ref, b_ref, o_ref):
        p = p_ref[...]
        w = w_ref[...]
        logits = lax.dot_general(
            p, w, dimension_numbers=(((1,), (1,)), ((), ())),
            preferred_element_type=jnp.float32)
        logits = logits + b_ref[...]
        m = jnp.max(logits, axis=1, keepdims=True)
        lse = m + jnp.log(jnp.sum(jnp.exp(logits - m), axis=1, keepdims=True))
        o_ref[...] = logits - lse

    return pl.pallas_call(
        body,
        out_shape=jax.ShapeDtypeStruct((BATCH, 2), jnp.float32),
    )(pooled, W, b.reshape(1, 2))


def kernel(text, table, W, b):
    table_flat = lax.optimization_barrier(table.reshape(-1))
    table_lin = table_flat.reshape(table.shape)
    pooled = _sc_pool(text.astype(jnp.int32), table_lin)
    return _tc_head(pooled, W, b)


# speedup vs baseline: 6.6618x; 1.0089x over previous
import functools

import numpy as np

import jax
import jax.numpy as jnp
from jax import lax
from jax.experimental import pallas as pl
from jax.experimental.pallas import tpu as pltpu
from jax.experimental.pallas import tpu_sc as plsc

EMBED_DIM = 64
SEQ = 200
BATCH = 4096
NUM_WORKERS = 32
ROWS_PER_W = BATCH // NUM_WORKERS
LANES = 16
DCH = EMBED_DIM // LANES
CHUNK0 = 128
CHUNK1 = SEQ - CHUNK0


NBUF = 4
HALF = 500000
TW = 2048


def _sc_pool(text, table):
    mesh = plsc.VectorSubcoreMesh(core_axis_name="c", subcore_axis_name="s")

    @functools.partial(
        pl.kernel,
        out_type=jax.ShapeDtypeStruct((BATCH, EMBED_DIM), jnp.float32),
        mesh=mesh,
        compiler_params=pltpu.CompilerParams(use_tc_tiling_on_sc=False),
        scratch_types=[
            pltpu.VMEM((ROWS_PER_W, SEQ), jnp.int32),
            pltpu.VMEM((NBUF, SEQ, EMBED_DIM), jnp.float32),
            pltpu.VMEM((ROWS_PER_W, EMBED_DIM), jnp.float32),
            [pltpu.SemaphoreType.DMA] * NBUF,
        ],
    )
    def k(text_hbm, table_hbm, out_hbm, idx_v, rows_v, out_v, sems):
        wid = lax.axis_index("s") * 2 + lax.axis_index("c")
        base = wid * ROWS_PER_W
        pltpu.sync_copy(text_hbm.at[pl.ds(base, ROWS_PER_W)], idx_v)

        lane = lax.iota(jnp.int32, LANES)

        def remap(v):
            return jnp.where(v < HALF, v * 2, (v - HALF) * 2 + 1)

        def remap_body(r, _):
            for c in range(SEQ // LANES):
                st = c * LANES
                idx_v[r, pl.ds(st, LANES)] = remap(idx_v[r, pl.ds(st, LANES)])
            st = SEQ - LANES
            v = idx_v[r, pl.ds(st, LANES)]
            idx_v[r, pl.ds(st, LANES)] = jnp.where(
                lane < (SEQ // LANES) * LANES - st, v, remap(v))
            return 0

        lax.fori_loop(0, ROWS_PER_W, remap_body, 0)

        scale = jnp.float32(1.0 / SEQ)

        def fire(r, s):
            pltpu.async_copy(
                table_hbm.at[idx_v.at[r, pl.ds(0, CHUNK0)]],
                rows_v.at[s, pl.ds(0, CHUNK0)], sems[s])
            pltpu.async_copy(
                table_hbm.at[idx_v.at[r, pl.ds(CHUNK0, CHUNK1)]],
                rows_v.at[s, pl.ds(CHUNK0, CHUNK1)], sems[s])

        def drain(s):
            pltpu.make_async_copy(
                table_hbm.at[pl.ds(0, SEQ)], rows_v.at[s], sems[s]).wait()

        def reduce_slot(s, r):
            def seq_body(j, acc):
                return tuple(
                    acc[k_] + rows_v[s, j, pl.ds(k_ * LANES, LANES)]
                    for k_ in range(DCH))

            zeros = tuple(jnp.zeros((LANES,), jnp.float32) for _ in range(DCH))
            acc = lax.fori_loop(0, SEQ, seq_body, zeros, unroll=2)
            for k_ in range(DCH):
                out_v[r, pl.ds(k_ * LANES, LANES)] = acc[k_] * scale

        for s in range(NBUF):
            fire(s, s)

        def group_body(g, _):
            rg = g * NBUF
            for s in range(NBUF):
                drain(s)
                reduce_slot(s, rg + s)
                fire(rg + s + NBUF, s)
            return 0

        lax.fori_loop(0, ROWS_PER_W // NBUF - 1, group_body, 0)
        rg = ROWS_PER_W - NBUF
        for s in range(NBUF):
            drain(s)
            reduce_slot(s, rg + s)

        pltpu.sync_copy(out_v, out_hbm.at[pl.ds(base, ROWS_PER_W)])

    return k(text, table)


def _tc_head(pooled, W, b):
    def body(p_ref, w_ref, b_ref, o_ref):
        p = p_ref[...]
        w = w_ref[...]
        logits = lax.dot_general(
            p, w, dimension_numbers=(((1,), (1,)), ((), ())),
            preferred_element_type=jnp.float32)
        logits = logits + b_ref[...]
        m = jnp.max(logits, axis=1, keepdims=True)
        lse = m + jnp.log(jnp.sum(jnp.exp(logits - m), axis=1, keepdims=True))
        o_ref[...] = logits - lse

    return pl.pallas_call(
        body,
        out_shape=jax.ShapeDtypeStruct((BATCH, 2), jnp.float32),
    )(pooled, W, b.reshape(1, 2))


def _perm_matrix():
    p = np.zeros((2 * EMBED_DIM, 2 * EMBED_DIM), np.float32)
    for d in range(2 * EMBED_DIM):
        s = 2 * d if d < EMBED_DIM else 2 * (d - EMBED_DIM) + 1
        p[s, d] = 1.0
    return jnp.asarray(p)


def _tc_repack(table):
    table_i = table.T.reshape(2 * EMBED_DIM, HALF)

    def body(x_ref, p_ref, o_ref):
        o_ref[...] = lax.dot_general(
            x_ref[...], p_ref[...],
            dimension_numbers=(((0,), (0,)), ((), ())),
            preferred_element_type=jnp.float32)

    n = -(-HALF // TW)
    repacked = pl.pallas_call(
        body,
        grid=(n,),
        in_specs=[
            pl.BlockSpec((2 * EMBED_DIM, TW), lambda i: (0, i)),
            pl.BlockSpec((2 * EMBED_DIM, 2 * EMBED_DIM), lambda i: (0, 0)),
        ],
        out_specs=pl.BlockSpec((TW, 2 * EMBED_DIM), lambda i: (i, 0)),
        out_shape=jax.ShapeDtypeStruct((HALF, 2 * EMBED_DIM), jnp.float32),
    )(table_i, _perm_matrix())
    return repacked.reshape(2 * HALF, EMBED_DIM)


def kernel(text, table, W, b):
    pooled = _sc_pool(text.astype(jnp.int32), _tc_repack(table))
    return _tc_head(pooled, W, b)
